# baseline (speedup 1.0000x reference)
import functools

import jax
import jax.numpy as jnp
from jax import lax
from jax.experimental import pallas as pl
from jax.experimental.pallas import tpu as pltpu
from jax.experimental.pallas import tpu_sc as plsc

N = 10000
E = 320000
D = 128

NC = 2
NS = 16
NW = NC * NS
NPAD = 10240
STRIPE = NPAD // NS

EPT = E // NW
CL = 160
CHT = EPT // CL
TAIL = EPT - CHT * CL
CLD = 200
NCHD = EPT // CLD

BLK = 640


@functools.lru_cache(maxsize=None)
def _build_sc_kernels():
    mesh = plsc.VectorSubcoreMesh(
        core_axis_name="c", subcore_axis_name="s",
        num_cores=NC, num_subcores=NS)

    @functools.partial(
        pl.kernel,
        out_type=jax.ShapeDtypeStruct((NC * NPAD,), jnp.float32),
        mesh=mesh,
        scratch_types=[
            pltpu.VMEM((EPT,), jnp.int32),
            pltpu.VMEM((CLD,), jnp.float32),
            pltpu.VMEM_SHARED((NPAD,), jnp.float32),
            pltpu.SemaphoreType.DMA,
        ],
    )
    def deg_kernel(dst_hbm, ones_hbm, zrow_hbm, out_hbm,
                   idx_v, ones_v, deg_sh, sem):
        c = lax.axis_index("c")
        s = lax.axis_index("s")
        w = c * NS + s
        r0 = s * STRIPE
        pltpu.sync_copy(ones_hbm, ones_v)
        pltpu.sync_copy(dst_hbm.at[pl.ds(w * EPT, EPT)], idx_v)
        pltpu.sync_copy(zrow_hbm.at[pl.ds(r0, STRIPE)],
                        deg_sh.at[pl.ds(r0, STRIPE)])
        plsc.subcore_barrier()

        def fire(k, carry):
            pltpu.async_copy(
                ones_v, deg_sh.at[idx_v.at[pl.ds(k * CLD, CLD)]], sem,
                add=True)
            return carry

        lax.fori_loop(0, NCHD, fire, 0)

        def drain(k, carry):
            pltpu.make_async_copy(
                ones_v, deg_sh.at[idx_v.at[pl.ds(k * CLD, CLD)]], sem).wait()
            return carry

        lax.fori_loop(0, NCHD, drain, 0)
        plsc.subcore_barrier()
        pltpu.sync_copy(deg_sh.at[pl.ds(r0, STRIPE)],
                        out_hbm.at[pl.ds(c * NPAD + r0, STRIPE)])

    @functools.partial(
        pl.kernel,
        out_type=(jax.ShapeDtypeStruct((NC * NPAD, D), jnp.float32),
                  jax.ShapeDtypeStruct((NC * NPAD,), jnp.float32)),
        mesh=mesh,
        scratch_types=[
            [pltpu.VMEM((CL,), jnp.int32) for _ in range(4)],
            [pltpu.VMEM((CL,), jnp.int32) for _ in range(4)],
            [pltpu.VMEM((CL, D), jnp.float32) for _ in range(2)],
            [pltpu.VMEM((CL,), jnp.float32) for _ in range(2)],
            pltpu.VMEM_SHARED((NPAD, D), jnp.float32),
            pltpu.VMEM_SHARED((NPAD,), jnp.float32),
            [pltpu.SemaphoreType.DMA for _ in range(4)],
            [pltpu.SemaphoreType.DMA for _ in range(2)],
            [pltpu.SemaphoreType.DMA for _ in range(2)],
            [pltpu.SemaphoreType.DMA for _ in range(2)],
            [pltpu.SemaphoreType.DMA for _ in range(2)],
        ],
    )
    def main_kernel(src_hbm, dst_hbm, hs_hbm, dinv_hbm, zmat_hbm, zrow_hbm,
                    acc_out, csum_out,
                    si, di, ubufs, dbufs, acc_sh, csum_sh,
                    isems, usems, dsems, asems, csems):
        c = lax.axis_index("c")
        s = lax.axis_index("s")
        w = c * NS + s
        r0 = s * STRIPE
        base = w * EPT

        def fire_idx(k, j):
            pltpu.async_copy(
                src_hbm.at[pl.ds(base + k * CL, CL)], si[j], isems[j])
            pltpu.async_copy(
                dst_hbm.at[pl.ds(base + k * CL, CL)], di[j], isems[j])

        def wait_idx(k, j):
            pltpu.make_async_copy(
                src_hbm.at[pl.ds(base + k * CL, CL)], si[j], isems[j]).wait()
            pltpu.make_async_copy(
                dst_hbm.at[pl.ds(base + k * CL, CL)], di[j], isems[j]).wait()

        def fire_gathers(j, b):
            pltpu.async_copy(hs_hbm.at[si[j]], ubufs[b], usems[b])
            pltpu.async_copy(dinv_hbm.at[di[j]], dbufs[b], dsems[b])

        def wait_gathers(j, b):
            pltpu.make_async_copy(hs_hbm.at[si[j]], ubufs[b], usems[b]).wait()
            pltpu.make_async_copy(
                dinv_hbm.at[di[j]], dbufs[b], dsems[b]).wait()

        def fire_scatters(j, b):
            pltpu.async_copy(ubufs[b], acc_sh.at[di[j]], asems[b], add=True)
            pltpu.async_copy(dbufs[b], csum_sh.at[si[j]], csems[b], add=True)

        def wait_scatters(j, b):
            pltpu.make_async_copy(ubufs[b], acc_sh.at[di[j]], asems[b]).wait()
            pltpu.make_async_copy(
                dbufs[b], csum_sh.at[si[j]], csems[b]).wait()

        for j in range(4):
            fire_idx(j, j)
        for b in range(2):
            wait_idx(b, b)
            fire_gathers(b, b)
        pltpu.sync_copy(zmat_hbm.at[pl.ds(r0, STRIPE)],
                        acc_sh.at[pl.ds(r0, STRIPE)])
        pltpu.sync_copy(zrow_hbm.at[pl.ds(r0, STRIPE)],
                        csum_sh.at[pl.ds(r0, STRIPE)])
        plsc.subcore_barrier()

        def quad(i, carry):
            for u in range(4):
                k = i * 4 + u
                j = u
                b = u % 2
                wait_gathers(j, b)
                fire_scatters(j, b)

                @pl.when(k + 2 < CHT)
                def _():
                    wait_scatters(j, b)
                    j2 = (u + 2) % 4
                    wait_idx(k + 2, j2)
                    fire_gathers(j2, b)

                    @pl.when(k + 4 < CHT)
                    def _():
                        fire_idx(k + 4, j)
            return carry

        lax.fori_loop(0, CHT // 4, quad, 0)
        wait_gathers(0, 0)
        fire_scatters(0, 0)
        wait_gathers(1, 1)
        fire_scatters(1, 1)
        toff = base + CHT * CL
        sit = si[2].at[pl.ds(0, TAIL)]
        dit = di[2].at[pl.ds(0, TAIL)]
        pltpu.sync_copy(src_hbm.at[pl.ds(toff, TAIL)], sit)
        pltpu.sync_copy(dst_hbm.at[pl.ds(toff, TAIL)], dit)
        wait_scatters(0, 0)
        ubt = ubufs[0].at[pl.ds(0, TAIL)]
        dbt = dbufs[0].at[pl.ds(0, TAIL)]
        pltpu.async_copy(hs_hbm.at[sit], ubt, usems[0]).wait()
        pltpu.async_copy(dinv_hbm.at[dit], dbt, dsems[0]).wait()
        pltpu.sync_copy(ubt, acc_sh.at[dit], add=True)
        pltpu.sync_copy(dbt, csum_sh.at[sit], add=True)
        wait_scatters(1, 1)
        plsc.subcore_barrier()
        pltpu.sync_copy(acc_sh.at[pl.ds(r0, STRIPE)],
                        acc_out.at[pl.ds(c * NPAD + r0, STRIPE)])
        pltpu.sync_copy(csum_sh.at[pl.ds(r0, STRIPE)],
                        csum_out.at[pl.ds(c * NPAD + r0, STRIPE)])

    return deg_kernel, main_kernel


def _hs_body(x_ref, w_ref, d_ref, out_ref):
    h = jnp.dot(x_ref[...], w_ref[...], preferred_element_type=jnp.float32)
    out_ref[...] = h * d_ref[...]


def _hs_call(x_pad, W1, dinv_col):
    return pl.pallas_call(
        _hs_body,
        grid=(NPAD // BLK,),
        in_specs=[
            pl.BlockSpec((BLK, D), lambda i: (i, 0)),
            pl.BlockSpec((D, D), lambda i: (0, 0)),
            pl.BlockSpec((BLK, 1), lambda i: (i, 0)),
        ],
        out_specs=pl.BlockSpec((BLK, D), lambda i: (i, 0)),
        out_shape=jax.ShapeDtypeStruct((NPAD, D), jnp.float32),
    )(x_pad, W1, dinv_col)


def _comb_body(a0_ref, a1_ref, hs_ref, d_ref, c_ref, b1_ref, w2_ref, b2_ref,
               out_ref, vacc_ref):
    i = pl.program_id(0)

    @pl.when(i == 0)
    def _():
        vacc_ref[...] = jnp.zeros((1, D), jnp.float32)

    a = a0_ref[...] + a1_ref[...] + hs_ref[...]
    r = jnp.maximum(d_ref[...] * a + b1_ref[...], 0.0)
    row = i * BLK + lax.broadcasted_iota(jnp.int32, (BLK, 1), 0)
    r = jnp.where(row < N, r, 0.0)
    vacc_ref[...] += jnp.sum(c_ref[...] * r, axis=0, keepdims=True)

    @pl.when(i == pl.num_programs(0) - 1)
    def _():
        g = jnp.dot(vacc_ref[...], w2_ref[...],
                    preferred_element_type=jnp.float32)
        out_ref[...] = g * (1.0 / N) + b2_ref[...]


def _comb_call(acc2, hs, dinv_col, cvec_col, b1_row, W2, b2_row):
    nblk = NPAD // BLK
    return pl.pallas_call(
        _comb_body,
        grid=(nblk,),
        in_specs=[
            pl.BlockSpec((BLK, D), lambda i: (i, 0)),
            pl.BlockSpec((BLK, D), lambda i: (i + NPAD // BLK, 0)),
            pl.BlockSpec((BLK, D), lambda i: (i, 0)),
            pl.BlockSpec((BLK, 1), lambda i: (i, 0)),
            pl.BlockSpec((BLK, 1), lambda i: (i, 0)),
            pl.BlockSpec((1, D), lambda i: (0, 0)),
            pl.BlockSpec((D, D), lambda i: (0, 0)),
            pl.BlockSpec((1, D), lambda i: (0, 0)),
        ],
        out_specs=pl.BlockSpec((1, D), lambda i: (0, 0)),
        out_shape=jax.ShapeDtypeStruct((1, D), jnp.float32),
        scratch_shapes=[pltpu.VMEM((1, D), jnp.float32)],
    )(acc2, acc2, hs, dinv_col, cvec_col, b1_row, W2, b2_row)


def kernel(x, edge_index, W1, b1, W2, b2):
    deg_kernel, main_kernel = _build_sc_kernels()

    src1d = edge_index[0]
    dst1d = edge_index[1]

    ones_vec = jnp.ones((CLD,), jnp.float32)
    zrow = jnp.zeros((NPAD,), jnp.float32)
    zmat = jnp.zeros((NPAD, D), jnp.float32)

    deg2 = deg_kernel(dst1d, ones_vec, zrow)
    deg = deg2[:NPAD] + deg2[NPAD:] + 1.0
    dinv = lax.rsqrt(deg)

    hs = _hs_call(x, W1, dinv.reshape(NPAD, 1))

    acc2, csum2 = main_kernel(src1d, dst1d, hs, dinv, zmat, zrow)

    csum = csum2[:NPAD] + csum2[NPAD:]
    cvec = dinv * (csum + dinv)
    cvec = jnp.where(jnp.arange(NPAD) < N, cvec, 0.0)

    return _comb_call(acc2, hs, dinv.reshape(NPAD, 1), cvec.reshape(NPAD, 1),
                      b1.reshape(1, D), W2, b2.reshape(1, D))


# speedup vs baseline: 1.0473x; 1.0473x over previous
import functools

import jax
import jax.numpy as jnp
from jax import lax
from jax.experimental import pallas as pl
from jax.experimental.pallas import tpu as pltpu
from jax.experimental.pallas import tpu_sc as plsc

N = 10000
E = 320000
D = 128

NC = 2
NS = 16
NW = NC * NS
NPAD = 10240
STRIPE = NPAD // NS

EPT = E // NW
CL = 160
CHT = EPT // CL
TAIL = EPT - CHT * CL
CLD = 200
NCHD = EPT // CLD

BLK = 640


@functools.lru_cache(maxsize=None)
def _build_sc_kernels():
    mesh = plsc.VectorSubcoreMesh(
        core_axis_name="c", subcore_axis_name="s",
        num_cores=NC, num_subcores=NS)

    @functools.partial(
        pl.kernel,
        out_type=jax.ShapeDtypeStruct((NC * NPAD,), jnp.float32),
        mesh=mesh,
        scratch_types=[
            pltpu.VMEM((EPT,), jnp.int32),
            pltpu.VMEM((CLD,), jnp.float32),
            pltpu.VMEM_SHARED((NPAD,), jnp.float32),
            pltpu.SemaphoreType.DMA,
        ],
    )
    def deg_kernel(dst_hbm, ones_hbm, zrow_hbm, out_hbm,
                   idx_v, ones_v, deg_sh, sem):
        c = lax.axis_index("c")
        s = lax.axis_index("s")
        w = c * NS + s
        r0 = s * STRIPE
        pltpu.sync_copy(ones_hbm, ones_v)
        pltpu.sync_copy(dst_hbm.at[pl.ds(w * EPT, EPT)], idx_v)
        pltpu.sync_copy(zrow_hbm.at[pl.ds(r0, STRIPE)],
                        deg_sh.at[pl.ds(r0, STRIPE)])
        plsc.subcore_barrier()

        def fire(k, carry):
            pltpu.async_copy(
                ones_v, deg_sh.at[idx_v.at[pl.ds(k * CLD, CLD)]], sem,
                add=True)
            return carry

        lax.fori_loop(0, NCHD, fire, 0)

        def drain(k, carry):
            pltpu.make_async_copy(
                ones_v, deg_sh.at[idx_v.at[pl.ds(k * CLD, CLD)]], sem).wait()
            return carry

        lax.fori_loop(0, NCHD, drain, 0)
        plsc.subcore_barrier()
        pltpu.sync_copy(deg_sh.at[pl.ds(r0, STRIPE)],
                        out_hbm.at[pl.ds(c * NPAD + r0, STRIPE)])

    @functools.partial(
        pl.kernel,
        out_type=(jax.ShapeDtypeStruct((NC * NPAD, D), jnp.float32),
                  jax.ShapeDtypeStruct((NC * NPAD,), jnp.float32)),
        mesh=mesh,
        scratch_types=[
            [pltpu.VMEM((CL,), jnp.int32) for _ in range(4)],
            [pltpu.VMEM((CL,), jnp.int32) for _ in range(4)],
            [pltpu.VMEM((CL, D), jnp.float32) for _ in range(2)],
            [pltpu.VMEM((CL,), jnp.float32) for _ in range(2)],
            pltpu.VMEM_SHARED((NPAD, D), jnp.float32),
            pltpu.VMEM_SHARED((NPAD,), jnp.float32),
            [pltpu.SemaphoreType.DMA for _ in range(4)],
            [pltpu.SemaphoreType.DMA for _ in range(2)],
            [pltpu.SemaphoreType.DMA for _ in range(2)],
            [pltpu.SemaphoreType.DMA for _ in range(2)],
            [pltpu.SemaphoreType.DMA for _ in range(2)],
        ],
    )
    def main_kernel(src_hbm, dst_hbm, hs_hbm, dinv_hbm, zmat_hbm, zrow_hbm,
                    acc_out, csum_out,
                    si, di, ubufs, dbufs, acc_sh, csum_sh,
                    isems, usems, dsems, asems, csems):
        c = lax.axis_index("c")
        s = lax.axis_index("s")
        w = c * NS + s
        r0 = s * STRIPE
        base = w * EPT

        def fire_idx(k, j):
            pltpu.async_copy(
                src_hbm.at[pl.ds(base + k * CL, CL)], si[j], isems[j])
            pltpu.async_copy(
                dst_hbm.at[pl.ds(base + k * CL, CL)], di[j], isems[j])

        def wait_idx(k, j):
            pltpu.make_async_copy(
                src_hbm.at[pl.ds(base + k * CL, CL)], si[j], isems[j]).wait()
            pltpu.make_async_copy(
                dst_hbm.at[pl.ds(base + k * CL, CL)], di[j], isems[j]).wait()

        def fire_gathers(j, b):
            pltpu.async_copy(hs_hbm.at[si[j]], ubufs[b], usems[b])
            pltpu.async_copy(dinv_hbm.at[di[j]], dbufs[b], dsems[b])

        def wait_gathers(j, b):
            pltpu.make_async_copy(hs_hbm.at[si[j]], ubufs[b], usems[b]).wait()
            pltpu.make_async_copy(
                dinv_hbm.at[di[j]], dbufs[b], dsems[b]).wait()

        def fire_scatters(j, b):
            pltpu.async_copy(ubufs[b], acc_sh.at[di[j]], asems[b], add=True)
            pltpu.async_copy(dbufs[b], csum_sh.at[si[j]], csems[b], add=True)

        def wait_scatters(j, b):
            pltpu.make_async_copy(ubufs[b], acc_sh.at[di[j]], asems[b]).wait()
            pltpu.make_async_copy(
                dbufs[b], csum_sh.at[si[j]], csems[b]).wait()

        for j in range(4):
            fire_idx(j, j)
        for b in range(2):
            wait_idx(b, b)
            fire_gathers(b, b)
        pltpu.sync_copy(zmat_hbm.at[pl.ds(r0, STRIPE)],
                        acc_sh.at[pl.ds(r0, STRIPE)])
        pltpu.sync_copy(zrow_hbm.at[pl.ds(r0, STRIPE)],
                        csum_sh.at[pl.ds(r0, STRIPE)])
        plsc.subcore_barrier()

        def quad(i, carry):
            for u in range(4):
                k = i * 4 + u
                j = u
                b = u % 2
                wait_gathers(j, b)
                fire_scatters(j, b)

                @pl.when(k + 2 < CHT)
                def _():
                    wait_scatters(j, b)
                    j2 = (u + 2) % 4
                    wait_idx(k + 2, j2)
                    fire_gathers(j2, b)

                    @pl.when(k + 4 < CHT)
                    def _():
                        fire_idx(k + 4, j)
            return carry

        lax.fori_loop(0, CHT // 4, quad, 0)
        wait_gathers(0, 0)
        fire_scatters(0, 0)
        wait_gathers(1, 1)
        fire_scatters(1, 1)
        toff = base + CHT * CL
        sit = si[2].at[pl.ds(0, TAIL)]
        dit = di[2].at[pl.ds(0, TAIL)]
        pltpu.sync_copy(src_hbm.at[pl.ds(toff, TAIL)], sit)
        pltpu.sync_copy(dst_hbm.at[pl.ds(toff, TAIL)], dit)
        wait_scatters(0, 0)
        ubt = ubufs[0].at[pl.ds(0, TAIL)]
        dbt = dbufs[0].at[pl.ds(0, TAIL)]
        pltpu.async_copy(hs_hbm.at[sit], ubt, usems[0]).wait()
        pltpu.async_copy(dinv_hbm.at[dit], dbt, dsems[0]).wait()
        pltpu.sync_copy(ubt, acc_sh.at[dit], add=True)
        pltpu.sync_copy(dbt, csum_sh.at[sit], add=True)
        wait_scatters(1, 1)
        plsc.subcore_barrier()
        pltpu.sync_copy(acc_sh.at[pl.ds(r0, STRIPE)],
                        acc_out.at[pl.ds(c * NPAD + r0, STRIPE)])
        pltpu.sync_copy(csum_sh.at[pl.ds(r0, STRIPE)],
                        csum_out.at[pl.ds(c * NPAD + r0, STRIPE)])

    return deg_kernel, main_kernel


def _hs_body(x_ref, w_ref, d_ref, out_ref):
    h = jnp.dot(x_ref[...], w_ref[...], preferred_element_type=jnp.float32)
    out_ref[...] = h * d_ref[...]


def _hs_call(x_pad, W1, dinv_col):
    return pl.pallas_call(
        _hs_body,
        grid=(NPAD // BLK,),
        in_specs=[
            pl.BlockSpec((BLK, D), lambda i: (i, 0)),
            pl.BlockSpec((D, D), lambda i: (0, 0)),
            pl.BlockSpec((BLK, 1), lambda i: (i, 0)),
        ],
        out_specs=pl.BlockSpec((BLK, D), lambda i: (i, 0)),
        out_shape=jax.ShapeDtypeStruct((NPAD, D), jnp.float32),
    )(x_pad, W1, dinv_col)


def _comb_body(a0_ref, a1_ref, hs_ref, d_ref, c_ref, b1_ref, w2_ref, b2_ref,
               out_ref, vacc_ref):
    i = pl.program_id(0)

    @pl.when(i == 0)
    def _():
        vacc_ref[...] = jnp.zeros((1, D), jnp.float32)

    a = a0_ref[...] + a1_ref[...] + hs_ref[...]
    r = jnp.maximum(d_ref[...] * a + b1_ref[...], 0.0)
    vacc_ref[...] += jnp.sum(c_ref[...] * r, axis=0, keepdims=True)

    @pl.when(i == pl.num_programs(0) - 1)
    def _():
        g = jnp.dot(vacc_ref[...], w2_ref[...],
                    preferred_element_type=jnp.float32)
        out_ref[...] = g * (1.0 / N) + b2_ref[...]


def _comb_call(acc2, hs, dinv_col, cvec_col, b1_row, W2, b2_row):
    nblk = NPAD // BLK
    return pl.pallas_call(
        _comb_body,
        grid=(nblk,),
        in_specs=[
            pl.BlockSpec((BLK, D), lambda i: (i, 0)),
            pl.BlockSpec((BLK, D), lambda i: (i + NPAD // BLK, 0)),
            pl.BlockSpec((BLK, D), lambda i: (i, 0)),
            pl.BlockSpec((BLK, 1), lambda i: (i, 0)),
            pl.BlockSpec((BLK, 1), lambda i: (i, 0)),
            pl.BlockSpec((1, D), lambda i: (0, 0)),
            pl.BlockSpec((D, D), lambda i: (0, 0)),
            pl.BlockSpec((1, D), lambda i: (0, 0)),
        ],
        out_specs=pl.BlockSpec((1, D), lambda i: (0, 0)),
        out_shape=jax.ShapeDtypeStruct((1, D), jnp.float32),
        scratch_shapes=[pltpu.VMEM((1, D), jnp.float32)],
    )(acc2, acc2, hs, dinv_col, cvec_col, b1_row, W2, b2_row)


def kernel(x, edge_index, W1, b1, W2, b2):
    deg_kernel, main_kernel = _build_sc_kernels()

    src1d = edge_index[0]
    dst1d = edge_index[1]
    x_pad = jnp.pad(x, ((0, NPAD - N), (0, 0)))

    ones_vec = jnp.ones((CLD,), jnp.float32)
    zrow = jnp.zeros((NPAD,), jnp.float32)
    zmat = jnp.zeros((NPAD, D), jnp.float32)

    deg2 = deg_kernel(dst1d, ones_vec, zrow)
    deg = deg2[:NPAD] + deg2[NPAD:] + 1.0
    dinv = lax.rsqrt(deg)

    hs = _hs_call(x_pad, W1, dinv.reshape(NPAD, 1))

    acc2, csum2 = main_kernel(src1d, dst1d, hs, dinv, zmat, zrow)

    csum = csum2[:NPAD] + csum2[NPAD:]
    cvec = dinv * (csum + dinv)
    cvec = jnp.where(jnp.arange(NPAD) < N, cvec, 0.0)

    return _comb_call(acc2, hs, dinv.reshape(NPAD, 1), cvec.reshape(NPAD, 1),
                      b1.reshape(1, D), W2, b2.reshape(1, D))
